# baseline (speedup 1.0000x reference)
import jax
import jax.numpy as jnp
from jax.experimental import pallas as pl
from jax.experimental.pallas import tpu as pltpu

H = 224
W = 224
CIN = 96
COUT = 192
HB = 56


SUB = 8


def _conv_body(x_ref, top_ref, bot_ref, w_ref, b_ref, o_ref, xs):
    k = pl.program_id(2)

    @pl.when(k == 0)
    def _fill():
        zc = jnp.zeros((HB + 2, CIN, 1), jnp.bfloat16)
        xs[:, 0, :, 0:1] = zc
        xs[:, 2, :, W - 1:W] = zc

        v = jnp.swapaxes(x_ref[0], 0, 1).astype(jnp.bfloat16)
        t = top_ref[0, 0].astype(jnp.bfloat16)
        u = bot_ref[0, 0].astype(jnp.bfloat16)
        xs[0, 0, :, 1:W] = t[:, 0:W - 1]
        xs[1:HB + 1, 0, :, 1:W] = v[:, :, 0:W - 1]
        xs[HB + 1, 0, :, 1:W] = u[:, 0:W - 1]
        xs[0, 1] = t
        xs[1:HB + 1, 1] = v
        xs[HB + 1, 1] = u
        xs[0, 2, :, 0:W - 1] = t[:, 1:W]
        xs[1:HB + 1, 2, :, 0:W - 1] = v[:, :, 1:W]
        xs[HB + 1, 2, :, 0:W - 1] = u[:, 1:W]

    wv = w_ref[...]
    b = jnp.broadcast_to(b_ref[...], (COUT, W))
    base = k * SUB
    for r in range(SUB):
        col = xs[pl.ds(base + r, 3)].reshape(9 * CIN, W)
        o_ref[0, :, r, :] = b + jnp.dot(wv, col,
                                        preferred_element_type=jnp.float32)


def kernel(input, weight, bias):
    n = input.shape[0]
    zrow = jnp.zeros((n, CIN, 1, W), jnp.float32)
    top = jnp.concatenate([zrow, input[:, :, HB - 1:H - 1:HB, :]], axis=2)
    bot = jnp.concatenate([input[:, :, HB:H:HB, :], zrow], axis=2)
    top = jnp.transpose(top, (0, 2, 1, 3))
    bot = jnp.transpose(bot, (0, 2, 1, 3))
    w = jnp.transpose(weight, (0, 2, 3, 1)).reshape(COUT, 9 * CIN)
    w = w.astype(jnp.bfloat16)
    b = bias.reshape(COUT, 1)

    halo_spec = pl.BlockSpec((1, 1, CIN, W), lambda i, j, k: (i, j, 0, 0))
    out = pl.pallas_call(
        _conv_body,
        grid=(n, H // HB, HB // SUB),
        in_specs=[
            pl.BlockSpec((1, CIN, HB, W), lambda i, j, k: (i, 0, j, 0)),
            halo_spec, halo_spec,
            pl.BlockSpec((COUT, 9 * CIN), lambda i, j, k: (0, 0)),
            pl.BlockSpec((COUT, 1), lambda i, j, k: (0, 0)),
        ],
        out_specs=pl.BlockSpec((1, COUT, SUB, W),
                               lambda i, j, k: (i, 0, j * (HB // SUB) + k, 0)),
        out_shape=jax.ShapeDtypeStruct((n, COUT, H, W), jnp.float32),
        scratch_shapes=[pltpu.VMEM((HB + 2, 3, CIN, W), jnp.bfloat16)],
    )(input, top, bot, w, b)
    return out


# speedup vs baseline: 1.3194x; 1.3194x over previous
import jax
import jax.numpy as jnp
from jax.experimental import pallas as pl
from jax.experimental.pallas import tpu as pltpu

H = 224
W = 224
CIN = 96
COUT = 192
HB = 56


def _conv_body(x_ref, top_ref, bot_ref, w_ref, b_ref, o_ref, xs):
    zc = jnp.zeros((HB + 2, CIN, 1), jnp.bfloat16)
    xs[:, 0, :, 0:1] = zc
    xs[:, 2, :, W - 1:W] = zc

    v = jnp.swapaxes(x_ref[0], 0, 1).astype(jnp.bfloat16)
    t = top_ref[0, 0].astype(jnp.bfloat16)
    u = bot_ref[0, 0].astype(jnp.bfloat16)
    xs[0, 0, :, 1:W] = t[:, 0:W - 1]
    xs[1:HB + 1, 0, :, 1:W] = v[:, :, 0:W - 1]
    xs[HB + 1, 0, :, 1:W] = u[:, 0:W - 1]
    xs[0, 1] = t
    xs[1:HB + 1, 1] = v
    xs[HB + 1, 1] = u
    xs[0, 2, :, 0:W - 1] = t[:, 1:W]
    xs[1:HB + 1, 2, :, 0:W - 1] = v[:, :, 1:W]
    xs[HB + 1, 2, :, 0:W - 1] = u[:, 1:W]

    wv = w_ref[...]
    b = jnp.broadcast_to(b_ref[...], (COUT, W))
    for r in range(HB):
        col = xs[r:r + 3].reshape(9 * CIN, W)
        o_ref[0, :, r, :] = b + jnp.dot(wv, col,
                                        preferred_element_type=jnp.float32)


def kernel(input, weight, bias):
    n = input.shape[0]
    zrow = jnp.zeros((n, CIN, 1, W), jnp.float32)
    top = jnp.concatenate([zrow, input[:, :, HB - 1:H - 1:HB, :]], axis=2)
    bot = jnp.concatenate([input[:, :, HB:H:HB, :], zrow], axis=2)
    top = jnp.transpose(top, (0, 2, 1, 3))
    bot = jnp.transpose(bot, (0, 2, 1, 3))
    w = jnp.transpose(weight, (0, 2, 3, 1)).reshape(COUT, 9 * CIN)
    w = w.astype(jnp.bfloat16)
    b = bias.reshape(COUT, 1)

    halo_spec = pl.BlockSpec((1, 1, CIN, W), lambda i, j: (i, j, 0, 0))
    out = pl.pallas_call(
        _conv_body,
        grid=(n, H // HB),
        in_specs=[
            pl.BlockSpec((1, CIN, HB, W), lambda i, j: (i, 0, j, 0)),
            halo_spec, halo_spec,
            pl.BlockSpec((COUT, 9 * CIN), lambda i, j: (0, 0)),
            pl.BlockSpec((COUT, 1), lambda i, j: (0, 0)),
        ],
        out_specs=pl.BlockSpec((1, COUT, HB, W), lambda i, j: (i, 0, j, 0)),
        out_shape=jax.ShapeDtypeStruct((n, COUT, H, W), jnp.float32),
        scratch_shapes=[pltpu.VMEM((HB + 2, 3, CIN, W), jnp.bfloat16)],
    )(input, top, bot, w, b)
    return out


# speedup vs baseline: 1.3249x; 1.0042x over previous
import jax
import jax.numpy as jnp
from jax.experimental import pallas as pl
from jax.experimental.pallas import tpu as pltpu

H = 224
W = 224
CIN = 96
COUT = 192
HB = 56


def _conv_body(x_ref, top_ref, bot_ref, w_ref, b_ref, o_ref, xs):
    zc = jnp.zeros((HB + 2, CIN, 1), jnp.bfloat16)
    xs[:, 0, :, 0:1] = zc
    xs[:, 2, :, W - 1:W] = zc

    v = jnp.swapaxes(x_ref[0], 0, 1).astype(jnp.bfloat16)
    t = top_ref[0, 0].astype(jnp.bfloat16)
    u = bot_ref[0, 0].astype(jnp.bfloat16)
    xs[0, 0, :, 1:W] = t[:, 0:W - 1]
    xs[1:HB + 1, 0, :, 1:W] = v[:, :, 0:W - 1]
    xs[HB + 1, 0, :, 1:W] = u[:, 0:W - 1]
    xs[0, 1] = t
    xs[1:HB + 1, 1] = v
    xs[HB + 1, 1] = u
    xs[0, 2, :, 0:W - 1] = t[:, 1:W]
    xs[1:HB + 1, 2, :, 0:W - 1] = v[:, :, 1:W]
    xs[HB + 1, 2, :, 0:W - 1] = u[:, 1:W]

    wv = w_ref[...]
    b = jnp.broadcast_to(b_ref[...], (COUT, W))
    for r in range(HB):
        col = xs[r:r + 3].reshape(9 * CIN, W)
        o_ref[0, :, r, :] = b + jnp.dot(wv, col,
                                        preferred_element_type=jnp.float32)


def kernel(input, weight, bias):
    n = input.shape[0]
    zrow = jnp.zeros((n, CIN, 1, W), jnp.float32)
    top = jnp.concatenate([zrow, input[:, :, HB - 1:H - 1:HB, :]], axis=2)
    bot = jnp.concatenate([input[:, :, HB:H:HB, :], zrow], axis=2)
    top = jnp.transpose(top, (0, 2, 1, 3))
    bot = jnp.transpose(bot, (0, 2, 1, 3))
    w = jnp.transpose(weight, (0, 2, 3, 1)).reshape(COUT, 9 * CIN)
    w = w.astype(jnp.bfloat16)
    b = bias.reshape(COUT, 1)

    halo_spec = pl.BlockSpec((1, 1, CIN, W), lambda i, j: (i, j, 0, 0))
    out = pl.pallas_call(
        _conv_body,
        grid=(n, H // HB),
        in_specs=[
            pl.BlockSpec((1, CIN, HB, W), lambda i, j: (i, 0, j, 0)),
            halo_spec, halo_spec,
            pl.BlockSpec((COUT, 9 * CIN), lambda i, j: (0, 0)),
            pl.BlockSpec((COUT, 1), lambda i, j: (0, 0)),
        ],
        out_specs=pl.BlockSpec((1, COUT, HB, W), lambda i, j: (i, 0, j, 0)),
        out_shape=jax.ShapeDtypeStruct((n, COUT, H, W), jnp.float32),
        scratch_shapes=[pltpu.VMEM((HB + 2, 3, CIN, W), jnp.bfloat16)],
        compiler_params=pltpu.CompilerParams(
            dimension_semantics=("parallel", "parallel")),
    )(input, top, bot, w, b)
    return out
